# baseline (speedup 1.0000x reference)
import functools

import jax
import jax.numpy as jnp
from jax import lax
from jax.experimental import pallas as pl
from jax.experimental.pallas import tpu as pltpu
from jax.experimental.pallas import tpu_sc as plsc

_N = 50000
_CIN = 128
_COUT = 16
_OUTS = (998, 998, 38)
_KVOL = 27
_M = _N * _KVOL
_BIG = _OUTS[0] * _OUTS[1] * _OUTS[2]
_NW = 32
_S = 128
_C = 42240
_EPAD = _NW * _C
_MALLOC = _M + _NW
_FR = 42240
_ROWBLK = 1000


def _mm_body(x_ref, w_ref, o_ref):
    o_ref[...] = jnp.dot(x_ref[...], w_ref[...],
                         preferred_element_type=jnp.float32)


def _matmul(x, wflat):
    return pl.pallas_call(
        _mm_body,
        grid=(_N // _ROWBLK,),
        in_specs=[
            pl.BlockSpec((_ROWBLK, _CIN), lambda i: (i, 0)),
            pl.BlockSpec((_CIN, _KVOL * _COUT), lambda i: (0, 0)),
        ],
        out_specs=pl.BlockSpec((_ROWBLK, _KVOL * _COUT), lambda i: (i, 0)),
        out_shape=jax.ShapeDtypeStruct((_N, _KVOL * _COUT), jnp.float32),
    )(x, wflat)


def _sc_body(nc,
             sk_hbm, rk_hbm, od_hbm, p_hbm, meta_hbm, bias_hbm,
             fillz_hbm, fillb_hbm,
             out_hbm, uniq_hbm,
             meta_v, bias_v, sk_v, rk_v, od_v, rows_v, acc_v, ukeys_v,
             idx_v, fillz_v, fillb_v,
             sem_g, sem_s1, sem_s2, sem_s3, sem_s4):
    wid = lax.axis_index("s") * nc + lax.axis_index("c")
    pltpu.sync_copy(meta_hbm.at[wid], meta_v)
    pltpu.sync_copy(bias_hbm, bias_v)
    pltpu.sync_copy(fillz_hbm, fillz_v)
    pltpu.sync_copy(fillb_hbm, fillb_v)

    mv = meta_v[...]
    j0 = mv[0]
    jend = mv[1]
    rlo = mv[2]
    rhi = mv[3]
    nsub = mv[4]
    u_cnt = mv[5]
    dump = _M + wid
    iota = lax.iota(jnp.int32, 16)
    zeros16i = jnp.zeros((16,), jnp.int32)
    biasvec = bias_v[...]

    def subchunk(k, carry):
        c_row, c_rank, c_key = carry
        jb = pl.multiple_of(j0 + k * _S, 8)
        cp_sk = pltpu.async_copy(sk_hbm.at[pl.ds(jb, _S + 16)], sk_v, sem_s3)
        cp_rk = pltpu.async_copy(rk_hbm.at[pl.ds(jb, _S + 16)], rk_v, sem_s4)
        pltpu.sync_copy(od_hbm.at[pl.ds(jb, _S)], od_v)
        cp_g = pltpu.async_copy(p_hbm.at[od_v], rows_v, sem_g)
        cp_sk.wait()
        cp_rk.wait()
        cp_g.wait()
        rbase = rk_v[pl.ds(0, 16)][0]

        @pl.when(k > 0)
        def _():
            pltpu.make_async_copy(acc_v, out_hbm.at[idx_v], sem_s1).wait()
            pltpu.make_async_copy(ukeys_v, uniq_hbm.at[idx_v], sem_s2).wait()

        cmv = jnp.broadcast_to(c_rank == rbase, (16,))
        plsc.store_scatter(acc_v, [zeros16i, iota], c_row, mask=cmv)
        plsc.store_scatter(ukeys_v, [zeros16i],
                           jnp.broadcast_to(c_key, (16,)), mask=cmv)

        def grp(g, prev_r):
            rk16 = rk_v[pl.ds(g * 16, 16)]
            sk16 = sk_v[pl.ds(g * 16, 16)]
            pr = prev_r
            for lane in range(16):
                r = rk16[lane]
                key = sk16[lane]
                newseg = r != pr
                key_ok = key < _BIG
                base_m = ((jb + g * 16 + lane) < jend) & (r >= rlo)
                lv = jnp.broadcast_to(r - rbase, (16,))
                rowvals = plsc.load_gather(
                    rows_v, [jnp.broadcast_to(g * 16 + lane, (16,)), iota])
                storev = jnp.where(jnp.broadcast_to(key_ok, (16,)),
                                   rowvals + biasvec, 0.0)
                sm = jnp.broadcast_to(base_m & newseg, (16,))
                am = jnp.broadcast_to(base_m & (~newseg) & key_ok, (16,))
                plsc.store_scatter(acc_v, [lv, iota], storev, mask=sm)
                plsc.addupdate_scatter(acc_v, [lv, iota], rowvals, mask=am)
                plsc.store_scatter(ukeys_v, [lv],
                                   jnp.broadcast_to(key, (16,)), mask=sm)
                pr = r
            return pr

        lax.fori_loop(0, _S // 16, grp, c_rank)

        tailr = rk_v[pl.ds(_S - 1, 16)]
        r_last = tailr[0]
        r_next = tailr[1]
        hi = jnp.where(k == nsub - 1, rhi,
                       jnp.where(r_next != r_last, r_last + 1, r_last))

        def bidx(i16, _):
            vals = rbase + i16 * 16 + iota
            m = (vals >= rlo) & (vals < hi)
            plsc.store_scatter(idx_v, [i16 * 16 + iota],
                               jnp.where(m, vals, dump))
            return 0

        lax.fori_loop(0, 8, bidx, 0)
        pltpu.async_copy(acc_v, out_hbm.at[idx_v], sem_s1)
        pltpu.async_copy(ukeys_v, uniq_hbm.at[idx_v], sem_s2)

        c_row2 = plsc.load_gather(
            acc_v, [jnp.broadcast_to(r_last - rbase, (16,)), iota])
        return (c_row2, r_last, sk_v[pl.ds(_S - 1, 16)][0])

    init = (jnp.zeros((16,), jnp.float32), jnp.int32(-1), jnp.int32(-1))
    lax.fori_loop(0, nsub, subchunk, init)

    @pl.when(nsub > 0)
    def _():
        pltpu.make_async_copy(acc_v, out_hbm.at[idx_v], sem_s1).wait()
        pltpu.make_async_copy(ukeys_v, uniq_hbm.at[idx_v], sem_s2).wait()

    fbase = wid * _FR

    def fill(u, _):
        base = fbase + u * _S

        @pl.when(~((base + _S <= u_cnt) | (base >= _M)))
        def _():
            def fidx(i16, _2):
                vals = base + i16 * 16 + iota
                m = (vals >= u_cnt) & (vals < _M)
                plsc.store_scatter(idx_v, [i16 * 16 + iota],
                                   jnp.where(m, vals, dump))
                return 0

            lax.fori_loop(0, 8, fidx, 0)
            cp_o = pltpu.async_copy(fillz_v, out_hbm.at[idx_v], sem_s1)
            cp_u = pltpu.async_copy(fillb_v, uniq_hbm.at[idx_v], sem_s2)
            cp_o.wait()
            cp_u.wait()

        return 0

    lax.fori_loop(0, _FR // _S, fill, 0)


def kernel(input, coords, W, bias):
    x = input.astype(jnp.float32)
    wflat = jnp.transpose(W.astype(jnp.float32), (1, 0, 2)).reshape(
        _CIN, _KVOL * _COUT)
    p = _matmul(x, wflat).reshape(_M, _COUT)

    sp = coords[:, 1:4].astype(jnp.int32)
    ar = jnp.arange(3, dtype=jnp.int32)
    offs = jnp.stack(jnp.meshgrid(ar, ar, ar, indexing="ij"),
                     axis=-1).reshape(_KVOL, 3)
    cand = sp[:, None, :] - offs[None, :, :]
    valid = jnp.all((cand >= 0) & (cand < jnp.array(_OUTS, jnp.int32)),
                    axis=-1)
    keys = (cand[..., 0] * _OUTS[1] + cand[..., 1]) * _OUTS[2] + cand[..., 2]
    keys = jnp.where(valid, keys, _BIG).reshape(-1).astype(jnp.int32)

    keys_p = jnp.concatenate(
        [keys, jnp.full((_EPAD - _M,), _BIG, jnp.int32)])
    pay = jnp.concatenate(
        [jnp.arange(_M, dtype=jnp.int32), jnp.zeros((_EPAD - _M,), jnp.int32)])
    sk, od = lax.sort((keys_p, pay), num_keys=1)
    ns = jnp.concatenate(
        [jnp.ones((1,), jnp.int32), (sk[1:] != sk[:-1]).astype(jnp.int32)])
    rk = jnp.cumsum(ns).astype(jnp.int32) - 1
    u_cnt = rk[-1]

    st = jnp.arange(_NW, dtype=jnp.int32) * _C
    cont = (st > 0) & (sk[st] == sk[jnp.maximum(st - 1, 0)])
    rlo = jnp.minimum(rk[st] + cont.astype(jnp.int32), u_cnt)
    rhi = jnp.minimum(jnp.concatenate([rlo[1:], u_cnt[None] + 1]), u_cnt)
    jskip = jnp.searchsorted(rk, rlo, side="left").astype(jnp.int32)
    jendx = jnp.searchsorted(rk, rhi, side="left").astype(jnp.int32)
    j0 = jskip & ~jnp.int32(7)
    nsub = jnp.where(rlo < rhi, (jendx - j0 + _S - 1) // _S, 0)

    meta = jnp.zeros((_NW, 16), jnp.int32)
    meta = meta.at[:, 0].set(j0)
    meta = meta.at[:, 1].set(jendx)
    meta = meta.at[:, 2].set(rlo)
    meta = meta.at[:, 3].set(rhi)
    meta = meta.at[:, 4].set(nsub)
    meta = meta.at[:, 5].set(u_cnt)

    ske = jnp.concatenate([sk, jnp.full((_S + 16,), _BIG, jnp.int32)])
    rke = jnp.concatenate([rk, jnp.broadcast_to(u_cnt, (_S + 16,))])
    ode = jnp.concatenate([od, jnp.zeros((_S + 16,), jnp.int32)])

    fillz = jnp.zeros((_S, _COUT), jnp.float32)
    fillb = jnp.full((_S,), _BIG, jnp.int32)

    info = plsc.get_sparse_core_info()
    nc = info.num_cores
    mesh = plsc.VectorSubcoreMesh(core_axis_name="c", subcore_axis_name="s")
    sc = pl.kernel(
        functools.partial(_sc_body, nc),
        out_type=[
            jax.ShapeDtypeStruct((_MALLOC, _COUT), jnp.float32),
            jax.ShapeDtypeStruct((_MALLOC,), jnp.int32),
        ],
        mesh=mesh,
        compiler_params=pltpu.CompilerParams(needs_layout_passes=False,
                                             use_tc_tiling_on_sc=False),
        scratch_types=[
            pltpu.VMEM((16,), jnp.int32),
            pltpu.VMEM((16,), jnp.float32),
            pltpu.VMEM((_S + 16,), jnp.int32),
            pltpu.VMEM((_S + 16,), jnp.int32),
            pltpu.VMEM((_S,), jnp.int32),
            pltpu.VMEM((_S, _COUT), jnp.float32),
            pltpu.VMEM((_S, _COUT), jnp.float32),
            pltpu.VMEM((_S,), jnp.int32),
            pltpu.VMEM((_S,), jnp.int32),
            pltpu.VMEM((_S, _COUT), jnp.float32),
            pltpu.VMEM((_S,), jnp.int32),
            pltpu.SemaphoreType.DMA,
            pltpu.SemaphoreType.DMA,
            pltpu.SemaphoreType.DMA,
            pltpu.SemaphoreType.DMA,
            pltpu.SemaphoreType.DMA,
        ],
    )
    out_full, uniq_full = sc(ske, rke, ode, p, meta,
                             bias.astype(jnp.float32), fillz, fillb)
    return out_full[:_M], uniq_full[:_M]


# speedup vs baseline: 1.0031x; 1.0031x over previous
import functools

import jax
import jax.numpy as jnp
from jax import lax
from jax.experimental import pallas as pl
from jax.experimental.pallas import tpu as pltpu
from jax.experimental.pallas import tpu_sc as plsc

_N = 50000
_CIN = 128
_COUT = 16
_OUTS = (998, 998, 38)
_KVOL = 27
_M = _N * _KVOL
_BIG = _OUTS[0] * _OUTS[1] * _OUTS[2]
_NW = 32
_S = 128
_C = 42240
_EPAD = _NW * _C
_MALLOC = _M + _NW
_FR = 42240
_ROWBLK = 1000


def _mm_body(x_ref, w_ref, o_ref):
    o_ref[...] = jnp.dot(x_ref[...], w_ref[...],
                         preferred_element_type=jnp.float32)


def _matmul(x, wflat):
    return pl.pallas_call(
        _mm_body,
        grid=(_N // _ROWBLK,),
        in_specs=[
            pl.BlockSpec((_ROWBLK, _CIN), lambda i: (i, 0)),
            pl.BlockSpec((_CIN, _KVOL * _COUT), lambda i: (0, 0)),
        ],
        out_specs=pl.BlockSpec((_ROWBLK, _KVOL * _COUT), lambda i: (i, 0)),
        out_shape=jax.ShapeDtypeStruct((_N, _KVOL * _COUT), jnp.float32),
    )(x, wflat)


def _sc_body(nc,
             sk_hbm, rk_hbm, od_hbm, p_hbm, meta_hbm, bias_hbm,
             fillz_hbm, fillb_hbm,
             out_hbm, uniq_hbm,
             meta_v, bias_v, sk_v, rk_v, od_v, rows_v, acc_v, ukeys_v,
             idx_v, fillz_v, fillb_v,
             sem_g, sem_s1, sem_s2, sem_s3, sem_s4):
    wid = lax.axis_index("s") * nc + lax.axis_index("c")
    pltpu.sync_copy(meta_hbm.at[wid], meta_v)
    pltpu.sync_copy(bias_hbm, bias_v)
    pltpu.sync_copy(fillz_hbm, fillz_v)
    pltpu.sync_copy(fillb_hbm, fillb_v)

    mv = meta_v[...]
    j0 = mv[0]
    jend = mv[1]
    rlo = mv[2]
    rhi = mv[3]
    nsub = mv[4]
    u_cnt = mv[5]
    bnz = mv[6]
    dump = _M + wid
    iota = lax.iota(jnp.int32, 16)
    zeros16i = jnp.zeros((16,), jnp.int32)
    zerorow = jnp.zeros((16,), jnp.float32)
    biasvec = bias_v[...]

    def subchunk(k, carry):
        c_row, c_rank, c_key = carry
        jb = pl.multiple_of(j0 + k * _S, 8)
        cp_sk = pltpu.async_copy(sk_hbm.at[pl.ds(jb, _S + 16)], sk_v, sem_s3)
        cp_rk = pltpu.async_copy(rk_hbm.at[pl.ds(jb, _S + 16)], rk_v, sem_s4)
        pltpu.sync_copy(od_hbm.at[pl.ds(jb, _S)], od_v)
        cp_g = pltpu.async_copy(p_hbm.at[od_v], rows_v, sem_g)
        cp_sk.wait()
        cp_rk.wait()
        cp_g.wait()
        rbase = rk_v[pl.ds(0, 16)][0]

        @pl.when(k > 0)
        def _():
            pltpu.make_async_copy(acc_v, out_hbm.at[idx_v], sem_s1).wait()
            pltpu.make_async_copy(ukeys_v, uniq_hbm.at[idx_v], sem_s2).wait()

        @plsc.parallel_loop(0, _S // 16, unroll=2)
        def _(g):
            for lane in range(16):
                plsc.store_scatter(
                    acc_v, [jnp.broadcast_to(g * 16 + lane, (16,)), iota],
                    zerorow)

        cmv = jnp.broadcast_to(c_rank == rbase, (16,))
        plsc.store_scatter(acc_v, [zeros16i, iota], c_row, mask=cmv)
        plsc.store_scatter(ukeys_v, [zeros16i],
                           jnp.broadcast_to(c_key, (16,)), mask=cmv)

        @plsc.parallel_loop(0, _S // 16, unroll=2)
        def _(g):
            rk16 = rk_v[pl.ds(g * 16, 16)]
            sk16 = sk_v[pl.ds(g * 16, 16)]
            for lane in range(16):
                r = rk16[lane]
                key = sk16[lane]
                base_m = ((jb + g * 16 + lane) < jend) & (r >= rlo)
                lv = jnp.broadcast_to(r - rbase, (16,))
                rowvals = plsc.load_gather(
                    rows_v, [jnp.broadcast_to(g * 16 + lane, (16,)), iota])
                am = jnp.broadcast_to(base_m & (key < _BIG), (16,))
                um = jnp.broadcast_to(base_m, (16,))
                plsc.addupdate_scatter(acc_v, [lv, iota], rowvals, mask=am)
                plsc.store_scatter(ukeys_v, [lv],
                                   jnp.broadcast_to(key, (16,)), mask=um)

        tailr = rk_v[pl.ds(_S - 1, 16)]
        r_last = tailr[0]
        r_next = tailr[1]
        hi = jnp.where(k == nsub - 1, rhi,
                       jnp.where(r_next != r_last, r_last + 1, r_last))

        c_row2 = plsc.load_gather(
            acc_v, [jnp.broadcast_to(r_last - rbase, (16,)), iota])

        def bidx(i16, _):
            vals = rbase + i16 * 16 + iota
            m = (vals >= rlo) & (vals < hi)
            plsc.store_scatter(idx_v, [i16 * 16 + iota],
                               jnp.where(m, vals, dump))
            return 0

        lax.fori_loop(0, 8, bidx, 0)

        @pl.when(bnz != 0)
        def _():
            @plsc.parallel_loop(0, _S // 16, unroll=2)
            def _(g):
                for lane in range(16):
                    plsc.addupdate_scatter(
                        acc_v, [jnp.broadcast_to(g * 16 + lane, (16,)), iota],
                        biasvec)

        pltpu.async_copy(acc_v, out_hbm.at[idx_v], sem_s1)
        pltpu.async_copy(ukeys_v, uniq_hbm.at[idx_v], sem_s2)

        return (c_row2, r_last, sk_v[pl.ds(_S - 1, 16)][0])

    init = (jnp.zeros((16,), jnp.float32), jnp.int32(-1), jnp.int32(-1))
    lax.fori_loop(0, nsub, subchunk, init)

    @pl.when(nsub > 0)
    def _():
        pltpu.make_async_copy(acc_v, out_hbm.at[idx_v], sem_s1).wait()
        pltpu.make_async_copy(ukeys_v, uniq_hbm.at[idx_v], sem_s2).wait()

    fbase = wid * _FR

    def fill(u, _):
        base = fbase + u * _S

        @pl.when(~((base + _S <= u_cnt) | (base >= _M)))
        def _():
            def fidx(i16, _2):
                vals = base + i16 * 16 + iota
                m = (vals >= u_cnt) & (vals < _M)
                plsc.store_scatter(idx_v, [i16 * 16 + iota],
                                   jnp.where(m, vals, dump))
                return 0

            lax.fori_loop(0, 8, fidx, 0)
            cp_o = pltpu.async_copy(fillz_v, out_hbm.at[idx_v], sem_s1)
            cp_u = pltpu.async_copy(fillb_v, uniq_hbm.at[idx_v], sem_s2)
            cp_o.wait()
            cp_u.wait()

        return 0

    lax.fori_loop(0, _FR // _S, fill, 0)


def kernel(input, coords, W, bias):
    x = input.astype(jnp.float32)
    wflat = jnp.transpose(W.astype(jnp.float32), (1, 0, 2)).reshape(
        _CIN, _KVOL * _COUT)
    p = _matmul(x, wflat).reshape(_M, _COUT)

    sp = coords[:, 1:4].astype(jnp.int32)
    ar = jnp.arange(3, dtype=jnp.int32)
    offs = jnp.stack(jnp.meshgrid(ar, ar, ar, indexing="ij"),
                     axis=-1).reshape(_KVOL, 3)
    cand = sp[:, None, :] - offs[None, :, :]
    valid = jnp.all((cand >= 0) & (cand < jnp.array(_OUTS, jnp.int32)),
                    axis=-1)
    keys = (cand[..., 0] * _OUTS[1] + cand[..., 1]) * _OUTS[2] + cand[..., 2]
    keys = jnp.where(valid, keys, _BIG).reshape(-1).astype(jnp.int32)

    keys_p = jnp.concatenate(
        [keys, jnp.full((_EPAD - _M,), _BIG, jnp.int32)])
    pay = jnp.concatenate(
        [jnp.arange(_M, dtype=jnp.int32), jnp.zeros((_EPAD - _M,), jnp.int32)])
    sk, od = lax.sort((keys_p, pay), num_keys=1)
    ns = jnp.concatenate(
        [jnp.ones((1,), jnp.int32), (sk[1:] != sk[:-1]).astype(jnp.int32)])
    rk = jnp.cumsum(ns).astype(jnp.int32) - 1
    u_cnt = rk[-1]

    st = jnp.arange(_NW, dtype=jnp.int32) * _C
    cont = (st > 0) & (sk[st] == sk[jnp.maximum(st - 1, 0)])
    rlo = jnp.minimum(rk[st] + cont.astype(jnp.int32), u_cnt)
    rhi = jnp.minimum(jnp.concatenate([rlo[1:], u_cnt[None] + 1]), u_cnt)
    jskip = jnp.searchsorted(rk, rlo, side="left").astype(jnp.int32)
    jendx = jnp.searchsorted(rk, rhi, side="left").astype(jnp.int32)
    j0 = jskip & ~jnp.int32(7)
    nsub = jnp.where(rlo < rhi, (jendx - j0 + _S - 1) // _S, 0)

    meta = jnp.zeros((_NW, 16), jnp.int32)
    meta = meta.at[:, 0].set(j0)
    meta = meta.at[:, 1].set(jendx)
    meta = meta.at[:, 2].set(rlo)
    meta = meta.at[:, 3].set(rhi)
    meta = meta.at[:, 4].set(nsub)
    meta = meta.at[:, 5].set(u_cnt)
    meta = meta.at[:, 6].set(jnp.any(bias != 0).astype(jnp.int32))

    ske = jnp.concatenate([sk, jnp.full((_S + 16,), _BIG, jnp.int32)])
    rke = jnp.concatenate([rk, jnp.broadcast_to(u_cnt, (_S + 16,))])
    ode = jnp.concatenate([od, jnp.zeros((_S + 16,), jnp.int32)])

    fillz = jnp.zeros((_S, _COUT), jnp.float32)
    fillb = jnp.full((_S,), _BIG, jnp.int32)

    info = plsc.get_sparse_core_info()
    nc = info.num_cores
    mesh = plsc.VectorSubcoreMesh(core_axis_name="c", subcore_axis_name="s")
    sc = pl.kernel(
        functools.partial(_sc_body, nc),
        out_type=[
            jax.ShapeDtypeStruct((_MALLOC, _COUT), jnp.float32),
            jax.ShapeDtypeStruct((_MALLOC,), jnp.int32),
        ],
        mesh=mesh,
        compiler_params=pltpu.CompilerParams(needs_layout_passes=False,
                                             use_tc_tiling_on_sc=False),
        scratch_types=[
            pltpu.VMEM((16,), jnp.int32),
            pltpu.VMEM((16,), jnp.float32),
            pltpu.VMEM((_S + 16,), jnp.int32),
            pltpu.VMEM((_S + 16,), jnp.int32),
            pltpu.VMEM((_S,), jnp.int32),
            pltpu.VMEM((_S, _COUT), jnp.float32),
            pltpu.VMEM((_S, _COUT), jnp.float32),
            pltpu.VMEM((_S,), jnp.int32),
            pltpu.VMEM((_S,), jnp.int32),
            pltpu.VMEM((_S, _COUT), jnp.float32),
            pltpu.VMEM((_S,), jnp.int32),
            pltpu.SemaphoreType.DMA,
            pltpu.SemaphoreType.DMA,
            pltpu.SemaphoreType.DMA,
            pltpu.SemaphoreType.DMA,
            pltpu.SemaphoreType.DMA,
        ],
    )
    out_full, uniq_full = sc(ske, rke, ode, p, meta,
                             bias.astype(jnp.float32), fillz, fillb)
    return out_full[:_M], uniq_full[:_M]
